# baseline (speedup 1.0000x reference)
import functools

import jax
import jax.numpy as jnp
from jax import lax
from jax.experimental import pallas as pl
from jax.experimental.pallas import tpu as pltpu
from jax.experimental.pallas import tpu_sc as plsc

NC = 2
NS = 16
NW = NC * NS
LANES = 16


def _bias_body(sym_t_ref, pos_t_ref, out_ref):
    colsum = jnp.sum(sym_t_ref[...], axis=1, keepdims=True)
    bias_t = pos_t_ref[...] - colsum * (1.0 / sym_t_ref.shape[1])
    out_ref[...] = jnp.transpose(bias_t)


def _repack_body(sym_t_ref, out_ref):
    out_ref[...] = jnp.transpose(sym_t_ref[...])


@functools.partial(jax.jit, static_argnames=("B", "S", "D"))
def _sc_embed(idx_t, sym_lin, bias, *, B, S, D):
    CB = 256
    NQ = B // CB
    NSLOT = 3
    base_p, rem = divmod(S, NW)
    max_p = base_p + (1 if rem else 0)
    nvec = D // LANES
    mesh = plsc.VectorSubcoreMesh(
        core_axis_name="c", subcore_axis_name="s", num_cores=NC, num_subcores=NS
    )

    @functools.partial(
        pl.kernel,
        out_type=jax.ShapeDtypeStruct((S, B, D), jnp.float32),
        mesh=mesh,
        scratch_types=[
            pltpu.VMEM((max_p, B), jnp.int32),
            pltpu.VMEM((max_p, D), jnp.float32),
            pltpu.VMEM((CB, D), jnp.float32),
            pltpu.VMEM((CB, D), jnp.float32),
            pltpu.VMEM((CB, D), jnp.float32),
            pltpu.SemaphoreType.DMA,
            pltpu.SemaphoreType.DMA,
            pltpu.SemaphoreType.DMA,
            pltpu.SemaphoreType.DMA,
            pltpu.SemaphoreType.DMA,
            pltpu.SemaphoreType.DMA,
        ],
        compiler_params=pltpu.CompilerParams(use_tc_tiling_on_sc=False),
    )
    def body(idx_hbm, sym_hbm, bias_hbm, out_hbm,
             idx_v, bias_v, rows0, rows1, rows2, g0, g1, g2, w0, w1, w2):
        wid = lax.axis_index("s") * NC + lax.axis_index("c")
        np_ = base_p + jnp.where(wid < rem, 1, 0)
        s0 = base_p * wid + jnp.minimum(wid, rem)
        nch = np_ * NQ
        start = jnp.minimum(s0, S - max_p)
        off = s0 - start
        rows = (rows0, rows1, rows2)
        gsem = (g0, g1, g2)
        wsem = (w0, w1, w2)

        pltpu.sync_copy(idx_hbm.at[pl.ds(start, max_p)], idx_v)
        pltpu.sync_copy(bias_hbm.at[pl.ds(start, max_p)], bias_v)

        def gather_args(c, slot):
            p, q = c // NQ, c % NQ
            return [
                (sym_hbm.at[idx_v.at[off + p, pl.ds(q * CB + j * 128, 128)]],
                 rows[slot].at[pl.ds(j * 128, 128)], gsem[slot])
                for j in range(CB // 128)
            ]

        def write_args(c, slot):
            p, q = c // NQ, c % NQ
            return (rows[slot], out_hbm.at[s0 + p, pl.ds(q * CB, CB)],
                    wsem[slot])

        def fire_gather(c, slot):
            for a in gather_args(c, slot):
                pltpu.async_copy(*a)

        def wait_gather(c, slot):
            for a in gather_args(c, slot):
                pltpu.make_async_copy(*a).wait()

        def compute(c, slot):
            p = c // NQ
            r_ref = rows[slot]
            UR = 8

            def r_loop(i, carry):
                r0 = i * UR
                for u in range(UR):
                    for v in range(nvec):
                        sl = pl.ds(v * LANES, LANES)
                        r_ref[r0 + u, sl] = r_ref[r0 + u, sl] + bias_v[off + p, sl]
                return carry

            lax.fori_loop(0, CB // UR, r_loop, 0)

        def chunk_iter(c, carry):
            for k in range(NSLOT):
                cc = NSLOT * c + k
                slot = (k + 1) % NSLOT

                @pl.when(cc < nch)
                def _():
                    @pl.when(cc + 1 < nch)
                    def _():
                        @pl.when(cc >= 2)
                        def _():
                            pltpu.make_async_copy(*write_args(cc - 2, slot)).wait()

                        fire_gather(cc + 1, slot)

                    wait_gather(cc, k)
                    compute(cc, k)
                    pltpu.async_copy(*write_args(cc, k))
            return carry

        fire_gather(0, 0)
        lax.fori_loop(0, (nch + NSLOT - 1) // NSLOT, chunk_iter, 0)
        for nch_s in (base_p * NQ, (base_p + 1) * NQ):

            @pl.when(nch == nch_s)
            def _():
                for c in (nch_s - 3, nch_s - 2, nch_s - 1):
                    pltpu.make_async_copy(*write_args(c, c % NSLOT)).wait()

    return body(idx_t, sym_lin, bias)


def kernel(inputs, sym_table, pos_table):
    B, S = inputs.shape
    V, D = sym_table.shape
    sym_t = sym_table.T
    pos_t = pos_table[:S].T
    idx_t = inputs.T.astype(jnp.int32)

    bias = pl.pallas_call(
        _bias_body,
        out_shape=jax.ShapeDtypeStruct((S, D), jnp.float32),
    )(sym_t, pos_t)

    BLK = 4096
    sym_lin = pl.pallas_call(
        _repack_body,
        out_shape=jax.ShapeDtypeStruct((V, D), jnp.float32),
        grid=(pl.cdiv(V, BLK),),
        in_specs=[pl.BlockSpec((D, BLK), lambda i: (0, i))],
        out_specs=pl.BlockSpec((BLK, D), lambda i: (i, 0)),
    )(sym_t)

    out_t = _sc_embed(idx_t, sym_lin, bias, B=B, S=S, D=D)
    return out_t.transpose(1, 0, 2)


# speedup vs baseline: 1.0873x; 1.0873x over previous
import functools

import jax
import jax.numpy as jnp
from jax import lax
from jax.experimental import pallas as pl
from jax.experimental.pallas import tpu as pltpu
from jax.experimental.pallas import tpu_sc as plsc

NC = 2
NS = 16
NW = NC * NS
LANES = 16


def _bias_body(sym_t_ref, pos_t_ref, out_ref):
    colsum = jnp.sum(sym_t_ref[...], axis=1, keepdims=True)
    bias_t = pos_t_ref[...] - colsum * (1.0 / sym_t_ref.shape[1])
    out_ref[...] = jnp.transpose(bias_t)


def _repack_body(sym_t_ref, out_ref):
    out_ref[...] = jnp.transpose(sym_t_ref[...])


@functools.partial(jax.jit, static_argnames=("B", "S", "D"))
def _sc_embed(idx_flat, sym_lin, bias, *, B, S, D):
    ROWS = B // NW
    C0 = 104
    C1 = S - C0
    mesh = plsc.VectorSubcoreMesh(
        core_axis_name="c", subcore_axis_name="s", num_cores=NC, num_subcores=NS
    )

    @functools.partial(
        pl.kernel,
        out_type=jax.ShapeDtypeStruct((B * S, D), jnp.float32),
        mesh=mesh,
        scratch_types=[
            pltpu.VMEM((S, D), jnp.float32),
            pltpu.VMEM((C0,), jnp.int32),
            pltpu.VMEM((C1,), jnp.int32),
            pltpu.VMEM((C0, D), jnp.float32),
            pltpu.VMEM((C1, D), jnp.float32),
            pltpu.SemaphoreType.DMA,
        ],
        compiler_params=pltpu.CompilerParams(use_tc_tiling_on_sc=False),
    )
    def body(idx_hbm, sym_hbm, bias_hbm, out_hbm, bias_v, idx0, idx1, rows0, rows1, sem):
        wid = lax.axis_index("s") * NC + lax.axis_index("c")
        pltpu.sync_copy(bias_hbm, bias_v)
        nvec = D // LANES

        def row_body(i, carry):
            base = (wid * ROWS + i) * S
            pltpu.sync_copy(idx_hbm.at[pl.ds(base, C0)], idx0)
            cp0 = pltpu.async_copy(sym_hbm.at[idx0], rows0, sem)
            pltpu.sync_copy(idx_hbm.at[pl.ds(base + C0, C1)], idx1)
            cp1 = pltpu.async_copy(sym_hbm.at[idx1], rows1, sem)
            cp0.wait()

            def add0(r, c2):
                for c in range(nvec):
                    sl = pl.ds(c * LANES, LANES)
                    rows0[r, sl] = rows0[r, sl] + bias_v[r, sl]
                return c2

            lax.fori_loop(0, C0, add0, 0)
            pltpu.sync_copy(rows0, out_hbm.at[pl.ds(base, C0)])
            cp1.wait()

            def add1(r, c2):
                for c in range(nvec):
                    sl = pl.ds(c * LANES, LANES)
                    rows1[r, sl] = rows1[r, sl] + bias_v[C0 + r, sl]
                return c2

            lax.fori_loop(0, C1, add1, 0)
            pltpu.sync_copy(rows1, out_hbm.at[pl.ds(base + C0, C1)])
            return carry

        lax.fori_loop(0, ROWS, row_body, 0)

    return body(idx_flat, sym_lin, bias)


def kernel(inputs, sym_table, pos_table):
    B, S = inputs.shape
    V, D = sym_table.shape
    sym_t = sym_table.T
    pos_t = pos_table[:S].T

    bias = pl.pallas_call(
        _bias_body,
        out_shape=jax.ShapeDtypeStruct((S, D), jnp.float32),
    )(sym_t, pos_t)

    BLK = 4096
    sym_lin = pl.pallas_call(
        _repack_body,
        out_shape=jax.ShapeDtypeStruct((V, D), jnp.float32),
        grid=(pl.cdiv(V, BLK),),
        in_specs=[pl.BlockSpec((D, BLK), lambda i: (0, i))],
        out_specs=pl.BlockSpec((BLK, D), lambda i: (i, 0)),
    )(sym_t)

    idx_flat = inputs.reshape(-1).astype(jnp.int32)
    out = _sc_embed(idx_flat, sym_lin, bias, B=B, S=S, D=D)
    return out.reshape(B, S, D)


# speedup vs baseline: 1.1118x; 1.0225x over previous
import functools

import jax
import jax.numpy as jnp
from jax import lax
from jax.experimental import pallas as pl
from jax.experimental.pallas import tpu as pltpu
from jax.experimental.pallas import tpu_sc as plsc

NC = 2
NS = 16
NW = NC * NS
LANES = 16


def _repack_bias_body(V, BLK, sym_t_ref, pos_t_ref, table_ref, bias_ref):
    i = pl.program_id(0)
    blk = sym_t_ref[...]
    table_ref[...] = jnp.transpose(blk)

    @pl.when(i == 0)
    def _():
        bias_ref[...] = jnp.transpose(pos_t_ref[...])

    col = jax.lax.broadcasted_iota(jnp.int32, blk.shape, 1)
    valid = col < (V - i * BLK)
    part = jnp.sum(jnp.where(valid, blk, 0.0), axis=1)
    bias_ref[...] = bias_ref[...] - part[None, :] * (1.0 / V)


@functools.partial(jax.jit, static_argnames=("B", "S", "D"))
def _sc_embed(idx_flat, sym_lin, bias, *, B, S, D):
    ROWS = B // NW
    C0 = 104
    C1 = S - C0
    mesh = plsc.VectorSubcoreMesh(
        core_axis_name="c", subcore_axis_name="s", num_cores=NC, num_subcores=NS
    )

    @functools.partial(
        pl.kernel,
        out_type=jax.ShapeDtypeStruct((B, S, D), jnp.float32),
        mesh=mesh,
        scratch_types=[
            pltpu.VMEM((S, D), jnp.float32),
            pltpu.VMEM((C0,), jnp.int32),
            pltpu.VMEM((C1,), jnp.int32),
            pltpu.VMEM((C0, D), jnp.float32),
            pltpu.VMEM((C1, D), jnp.float32),
            pltpu.SemaphoreType.DMA,
        ],
        compiler_params=pltpu.CompilerParams(use_tc_tiling_on_sc=False),
    )
    def body(idx_hbm, sym_hbm, bias_hbm, out_hbm, bias_v, idx0, idx1, rows0, rows1, sem):
        wid = lax.axis_index("s") * NC + lax.axis_index("c")
        pltpu.sync_copy(bias_hbm, bias_v)
        nvec = D // LANES

        def row_body(i, carry):
            b = wid * ROWS + i
            base = b * S
            pltpu.sync_copy(idx_hbm.at[pl.ds(base, C0)], idx0)
            cp0 = pltpu.async_copy(sym_hbm.at[idx0], rows0, sem)
            pltpu.sync_copy(idx_hbm.at[pl.ds(base + C0, C1)], idx1)
            cp1 = pltpu.async_copy(sym_hbm.at[idx1], rows1, sem)
            cp0.wait()

            def add0(r, c2):
                for c in range(nvec):
                    sl = pl.ds(c * LANES, LANES)
                    rows0[r, sl] = rows0[r, sl] + bias_v[r, sl]
                return c2

            lax.fori_loop(0, C0, add0, 0)
            pltpu.sync_copy(rows0, out_hbm.at[b, pl.ds(0, C0)])
            cp1.wait()

            def add1(r, c2):
                for c in range(nvec):
                    sl = pl.ds(c * LANES, LANES)
                    rows1[r, sl] = rows1[r, sl] + bias_v[C0 + r, sl]
                return c2

            lax.fori_loop(0, C1, add1, 0)
            pltpu.sync_copy(rows1, out_hbm.at[b, pl.ds(C0, C1)])
            return carry

        lax.fori_loop(0, ROWS, row_body, 0)

    return body(idx_flat, sym_lin, bias)


def kernel(inputs, sym_table, pos_table):
    B, S = inputs.shape
    V, D = sym_table.shape
    sym_t = sym_table.T
    pos_t = pos_table[:S].T

    BLK = 4096
    sym_lin, bias = pl.pallas_call(
        functools.partial(_repack_bias_body, V, BLK),
        out_shape=[
            jax.ShapeDtypeStruct((V, D), jnp.float32),
            jax.ShapeDtypeStruct((S, D), jnp.float32),
        ],
        grid=(pl.cdiv(V, BLK),),
        in_specs=[
            pl.BlockSpec((D, BLK), lambda i: (0, i)),
            pl.BlockSpec((D, S), lambda i: (0, 0)),
        ],
        out_specs=[
            pl.BlockSpec((BLK, D), lambda i: (i, 0)),
            pl.BlockSpec((S, D), lambda i: (0, 0)),
        ],
    )(sym_t, pos_t)

    idx_flat = inputs.reshape(-1).astype(jnp.int32)
    return _sc_embed(idx_flat, sym_lin, bias, B=B, S=S, D=D)


# speedup vs baseline: 1.2208x; 1.0981x over previous
import functools

import jax
import jax.numpy as jnp
from jax import lax
from jax.experimental import pallas as pl
from jax.experimental.pallas import tpu as pltpu
from jax.experimental.pallas import tpu_sc as plsc

NC = 2
NS = 16
NW = NC * NS
LANES = 16


def _repack_bias_body(V, BLK, sym_t_ref, pos_t_ref, table_ref, bias_ref):
    i = pl.program_id(0)
    blk = sym_t_ref[...]
    table_ref[...] = jnp.transpose(blk)

    @pl.when(i == 0)
    def _():
        bias_ref[...] = jnp.transpose(pos_t_ref[...])

    col = jax.lax.broadcasted_iota(jnp.int32, blk.shape, 1)
    valid = col < (V - i * BLK)
    part = jnp.sum(jnp.where(valid, blk, 0.0), axis=1)
    bias_ref[...] = bias_ref[...] - part[None, :] * (1.0 / V)


@functools.partial(jax.jit, static_argnames=("B", "S", "D"))
def _sc_embed(idx_flat, sym_lin, bias, *, B, S, D):
    SPAN = B * S // NW
    CH = 128
    NCHUNK = SPAN // CH
    NBUF = 3
    nvec = D // LANES
    mesh = plsc.VectorSubcoreMesh(
        core_axis_name="c", subcore_axis_name="s", num_cores=NC, num_subcores=NS
    )

    scratch = [pltpu.VMEM((2 * S, D), jnp.float32)]
    scratch += [pltpu.VMEM((CH,), jnp.int32) for _ in range(NBUF)]
    scratch += [pltpu.VMEM((CH, D), jnp.float32) for _ in range(NBUF)]
    scratch += [pltpu.SemaphoreType.DMA for _ in range(2 * NBUF)]

    @functools.partial(
        pl.kernel,
        out_type=jax.ShapeDtypeStruct((B * S, D), jnp.float32),
        mesh=mesh,
        scratch_types=scratch,
        compiler_params=pltpu.CompilerParams(use_tc_tiling_on_sc=False),
    )
    def body(idx_hbm, sym_hbm, bias_hbm, out_hbm, bias_v, *bufs):
        idxb = bufs[0:NBUF]
        rowsb = bufs[NBUF:2 * NBUF]
        gsem = bufs[2 * NBUF:3 * NBUF]
        wsem = bufs[3 * NBUF:4 * NBUF]
        wid = lax.axis_index("s") * NC + lax.axis_index("c")
        base0 = wid * SPAN
        pltpu.sync_copy(bias_hbm, bias_v.at[pl.ds(0, S)])
        pltpu.sync_copy(bias_hbm, bias_v.at[pl.ds(S, S)])

        gcp = [None] * NCHUNK
        wcp = [None] * NCHUNK

        def start(k):
            b = k % NBUF
            if k >= NBUF:
                wcp[k - NBUF].wait()
            off = base0 + k * CH
            pltpu.sync_copy(idx_hbm.at[pl.ds(off, CH)], idxb[b])
            gcp[k] = pltpu.async_copy(sym_hbm.at[idxb[b]], rowsb[b], gsem[b])

        def finish(k):
            b = k % NBUF
            gcp[k].wait()
            s0 = (k * CH) % S

            def add(r, c2):
                for c in range(nvec):
                    sl = pl.ds(c * LANES, LANES)
                    rowsb[b][r, sl] = rowsb[b][r, sl] + bias_v[s0 + r, sl]
                return c2

            lax.fori_loop(0, CH, add, 0)
            off = base0 + k * CH
            wcp[k] = pltpu.async_copy(rowsb[b], out_hbm.at[pl.ds(off, CH)], wsem[b])

        start(0)
        for k in range(1, NCHUNK):
            start(k)
            finish(k - 1)
        finish(NCHUNK - 1)
        for k in range(NCHUNK - NBUF, NCHUNK):
            wcp[k].wait()

    return body(idx_flat, sym_lin, bias)


def kernel(inputs, sym_table, pos_table):
    B, S = inputs.shape
    V, D = sym_table.shape
    sym_t = sym_table.T
    pos_t = pos_table[:S].T

    BLK = 4096
    sym_lin, bias = pl.pallas_call(
        functools.partial(_repack_bias_body, V, BLK),
        out_shape=[
            jax.ShapeDtypeStruct((V, D), jnp.float32),
            jax.ShapeDtypeStruct((S, D), jnp.float32),
        ],
        grid=(pl.cdiv(V, BLK),),
        in_specs=[
            pl.BlockSpec((D, BLK), lambda i: (0, i)),
            pl.BlockSpec((D, S), lambda i: (0, 0)),
        ],
        out_specs=[
            pl.BlockSpec((BLK, D), lambda i: (i, 0)),
            pl.BlockSpec((S, D), lambda i: (0, 0)),
        ],
    )(sym_t, pos_t)

    idx_flat = inputs.reshape(-1).astype(jnp.int32)
    out = _sc_embed(idx_flat, sym_lin, bias, B=B, S=S, D=D)
    return out.reshape(B, S, D)


# speedup vs baseline: 1.2243x; 1.0029x over previous
import functools

import jax
import jax.numpy as jnp
from jax import lax
from jax.experimental import pallas as pl
from jax.experimental.pallas import tpu as pltpu
from jax.experimental.pallas import tpu_sc as plsc

NC = 2
NS = 16
NW = NC * NS
LANES = 16


def _repack_bias_body(V, BLK, sym_t_ref, pos_t_ref, table_ref, bias_ref):
    i = pl.program_id(0)
    blk = sym_t_ref[...]
    table_ref[...] = jnp.transpose(blk)

    @pl.when(i == 0)
    def _():
        bias_ref[...] = jnp.transpose(pos_t_ref[...])

    col = jax.lax.broadcasted_iota(jnp.int32, blk.shape, 1)
    valid = col < (V - i * BLK)
    part = jnp.sum(jnp.where(valid, blk, 0.0), axis=1)
    bias_ref[...] = bias_ref[...] - part[None, :] * (1.0 / V)


@functools.partial(jax.jit, static_argnames=("B", "S", "D"))
def _sc_embed(idx_flat, sym_lin, bias, *, B, S, D):
    SPAN = B * S // NW
    CH = 128
    NCHUNK = SPAN // CH
    NBUF = 4
    nvec = D // LANES
    mesh = plsc.VectorSubcoreMesh(
        core_axis_name="c", subcore_axis_name="s", num_cores=NC, num_subcores=NS
    )

    scratch = [pltpu.VMEM((2 * S, D), jnp.float32)]
    scratch += [pltpu.VMEM((CH,), jnp.int32) for _ in range(NBUF)]
    scratch += [pltpu.VMEM((CH, D), jnp.float32) for _ in range(NBUF)]
    scratch += [pltpu.SemaphoreType.DMA for _ in range(2 * NBUF)]

    @functools.partial(
        pl.kernel,
        out_type=jax.ShapeDtypeStruct((B * S, D), jnp.float32),
        mesh=mesh,
        scratch_types=scratch,
        compiler_params=pltpu.CompilerParams(use_tc_tiling_on_sc=False),
    )
    def body(idx_hbm, sym_hbm, bias_hbm, out_hbm, bias_v, *bufs):
        idxb = bufs[0:NBUF]
        rowsb = bufs[NBUF:2 * NBUF]
        gsem = bufs[2 * NBUF:3 * NBUF]
        wsem = bufs[3 * NBUF:4 * NBUF]
        wid = lax.axis_index("s") * NC + lax.axis_index("c")
        base0 = wid * SPAN
        pltpu.sync_copy(bias_hbm, bias_v.at[pl.ds(0, S)])
        pltpu.sync_copy(bias_hbm, bias_v.at[pl.ds(S, S)])

        gcp = [None] * NCHUNK
        wcp = [None] * NCHUNK

        def start(k):
            b = k % NBUF
            if k >= NBUF:
                wcp[k - NBUF].wait()
            off = base0 + k * CH
            pltpu.sync_copy(idx_hbm.at[pl.ds(off, CH)], idxb[b])
            gcp[k] = pltpu.async_copy(sym_hbm.at[idxb[b]], rowsb[b], gsem[b])

        def finish(k):
            b = k % NBUF
            gcp[k].wait()
            s0 = (k * CH) % S

            def add(r, c2):
                for c in range(nvec):
                    sl = pl.ds(c * LANES, LANES)
                    rowsb[b][r, sl] = rowsb[b][r, sl] + bias_v[s0 + r, sl]
                return c2

            lax.fori_loop(0, CH, add, 0)
            off = base0 + k * CH
            wcp[k] = pltpu.async_copy(rowsb[b], out_hbm.at[pl.ds(off, CH)], wsem[b])

        start(0)
        for k in range(1, NCHUNK):
            start(k)
            finish(k - 1)
        finish(NCHUNK - 1)
        for k in range(NCHUNK - NBUF, NCHUNK):
            wcp[k].wait()

    return body(idx_flat, sym_lin, bias)


def kernel(inputs, sym_table, pos_table):
    B, S = inputs.shape
    V, D = sym_table.shape
    sym_t = sym_table.T
    pos_t = pos_table[:S].T

    BLK = 4096
    sym_lin, bias = pl.pallas_call(
        functools.partial(_repack_bias_body, V, BLK),
        out_shape=[
            jax.ShapeDtypeStruct((V, D), jnp.float32),
            jax.ShapeDtypeStruct((S, D), jnp.float32),
        ],
        grid=(pl.cdiv(V, BLK),),
        in_specs=[
            pl.BlockSpec((D, BLK), lambda i: (0, i)),
            pl.BlockSpec((D, S), lambda i: (0, 0)),
        ],
        out_specs=[
            pl.BlockSpec((BLK, D), lambda i: (i, 0)),
            pl.BlockSpec((S, D), lambda i: (0, 0)),
        ],
    )(sym_t, pos_t)

    idx_flat = inputs.reshape(-1).astype(jnp.int32)
    out = _sc_embed(idx_flat, sym_lin, bias, B=B, S=S, D=D)
    return out.reshape(B, S, D)


# speedup vs baseline: 1.2848x; 1.0494x over previous
import functools

import jax
import jax.numpy as jnp
from jax import lax
from jax.experimental import pallas as pl
from jax.experimental.pallas import tpu as pltpu
from jax.experimental.pallas import tpu_sc as plsc

NC = 2
NS = 16
NW = NC * NS
LANES = 16


def _repack_bias_body(V, BLK, sym_t_ref, pos_t_ref, table_ref, bias_ref):
    i = pl.program_id(0)
    blk = sym_t_ref[...]
    table_ref[...] = jnp.transpose(blk)

    @pl.when(i == 0)
    def _():
        bias_ref[...] = jnp.transpose(pos_t_ref[...])

    col = jax.lax.broadcasted_iota(jnp.int32, blk.shape, 1)
    valid = col < (V - i * BLK)
    part = jnp.sum(jnp.where(valid, blk, 0.0), axis=1)
    bias_ref[...] = bias_ref[...] - part[None, :] * (1.0 / V)


@functools.partial(jax.jit, static_argnames=("B", "S", "D"))
def _sc_embed(idx_flat, sym_lin, bias, *, B, S, D):
    SPAN = B * S // NW
    CH = 128
    NCHUNK = SPAN // CH
    NBUF = 4
    nvec = D // LANES
    mesh = plsc.VectorSubcoreMesh(
        core_axis_name="c", subcore_axis_name="s", num_cores=NC, num_subcores=NS
    )

    scratch = [pltpu.VMEM((2 * S, D), jnp.float32)]
    scratch += [pltpu.VMEM((CH,), jnp.int32) for _ in range(NBUF)]
    scratch += [pltpu.VMEM((CH, D), jnp.float32) for _ in range(NBUF)]
    scratch += [pltpu.SemaphoreType.DMA for _ in range(3 * NBUF)]

    @functools.partial(
        pl.kernel,
        out_type=jax.ShapeDtypeStruct((B * S, D), jnp.float32),
        mesh=mesh,
        scratch_types=scratch,
        compiler_params=pltpu.CompilerParams(use_tc_tiling_on_sc=False),
    )
    def body(idx_hbm, sym_hbm, bias_hbm, out_hbm, bias_v, *bufs):
        idxb = bufs[0:NBUF]
        rowsb = bufs[NBUF:2 * NBUF]
        gsem = bufs[2 * NBUF:3 * NBUF]
        wsem = bufs[3 * NBUF:4 * NBUF]
        isem = bufs[4 * NBUF:5 * NBUF]
        wid = lax.axis_index("s") * NC + lax.axis_index("c")
        base0 = wid * SPAN
        pltpu.sync_copy(bias_hbm, bias_v.at[pl.ds(0, S)])
        pltpu.sync_copy(bias_hbm, bias_v.at[pl.ds(S, S)])

        gcp = [None] * NCHUNK
        wcp = [None] * NCHUNK
        icp = [None] * NCHUNK

        def load_idx(k):
            b = k % NBUF
            off = base0 + k * CH
            icp[k] = pltpu.async_copy(idx_hbm.at[pl.ds(off, CH)], idxb[b], isem[b])

        def start(k):
            b = k % NBUF
            if k >= NBUF:
                wcp[k - NBUF].wait()
            icp[k].wait()
            gcp[k] = pltpu.async_copy(sym_hbm.at[idxb[b]], rowsb[b], gsem[b])
            if k + 1 < NCHUNK:
                load_idx(k + 1)

        def finish(k):
            b = k % NBUF
            gcp[k].wait()
            s0 = (k * CH) % S

            def add(r, c2):
                for c in range(nvec):
                    sl = pl.ds(c * LANES, LANES)
                    rowsb[b][r, sl] = rowsb[b][r, sl] + bias_v[s0 + r, sl]
                return c2

            lax.fori_loop(0, CH, add, 0)
            off = base0 + k * CH
            wcp[k] = pltpu.async_copy(rowsb[b], out_hbm.at[pl.ds(off, CH)], wsem[b])

        load_idx(0)
        start(0)
        for k in range(1, NCHUNK):
            start(k)
            finish(k - 1)
        finish(NCHUNK - 1)
        for k in range(NCHUNK - NBUF, NCHUNK):
            wcp[k].wait()

    return body(idx_flat, sym_lin, bias)


def kernel(inputs, sym_table, pos_table):
    B, S = inputs.shape
    V, D = sym_table.shape
    sym_t = sym_table.T
    pos_t = pos_table[:S].T

    BLK = 4096
    sym_lin, bias = pl.pallas_call(
        functools.partial(_repack_bias_body, V, BLK),
        out_shape=[
            jax.ShapeDtypeStruct((V, D), jnp.float32),
            jax.ShapeDtypeStruct((S, D), jnp.float32),
        ],
        grid=(pl.cdiv(V, BLK),),
        in_specs=[
            pl.BlockSpec((D, BLK), lambda i: (0, i)),
            pl.BlockSpec((D, S), lambda i: (0, 0)),
        ],
        out_specs=[
            pl.BlockSpec((BLK, D), lambda i: (i, 0)),
            pl.BlockSpec((S, D), lambda i: (0, 0)),
        ],
    )(sym_t, pos_t)

    idx_flat = inputs.reshape(-1).astype(jnp.int32)
    out = _sc_embed(idx_flat, sym_lin, bias, B=B, S=S, D=D)
    return out.reshape(B, S, D)
